# baseline (speedup 1.0000x reference)
import functools

import jax
import jax.numpy as jnp
from jax import lax
from jax.experimental import pallas as pl
from jax.experimental.pallas import tpu as pltpu
from jax.experimental.pallas import tpu_sc as plsc


def _ln(h, g, b):
    m = jnp.mean(h, axis=-1, keepdims=True)
    v = jnp.var(h, axis=-1, keepdims=True)
    return (h - m) / jnp.sqrt(v + 1e-5) * g + b


def _silu(h):
    return h * jax.nn.sigmoid(h)


def _tables_body(cb_ref, cm_w1_ref, cm_b1_ref, cm_g1_ref, cm_be1_ref,
                 cm_w2_ref, cm_b2_ref, cm_g2_ref, cm_be2_ref,
                 cm_w3_ref, cm_b3_ref, enc_w_ref, enc_b_ref,
                 dec_w1_ref, dec_b1_ref, dec_w2_ref, dec_b2_ref,
                 dec_w3_ref, dec_b3_ref,
                 lc_ref, te_ref, td_ref, c2_ref):
    cb = cb_ref[...]
    h = jnp.dot(cb, cm_w1_ref[...], preferred_element_type=jnp.float32)
    h = _silu(_ln(h + cm_b1_ref[...], cm_g1_ref[...], cm_be1_ref[...]))
    h = jnp.dot(h, cm_w2_ref[...], preferred_element_type=jnp.float32)
    h = _silu(_ln(h + cm_b2_ref[...], cm_g2_ref[...], cm_be2_ref[...]))
    lc = jnp.dot(h, cm_w3_ref[...], preferred_element_type=jnp.float32)
    lc = lc + cm_b3_ref[...]
    te = jnp.dot(lc, enc_w_ref[...], preferred_element_type=jnp.float32)
    te = te + enc_b_ref[...]
    hd = _silu(jnp.dot(te, dec_w1_ref[...], preferred_element_type=jnp.float32)
               + dec_b1_ref[...])
    hd = _silu(jnp.dot(hd, dec_w2_ref[...], preferred_element_type=jnp.float32)
               + dec_b2_ref[...])
    td = jnp.dot(hd, dec_w3_ref[...], preferred_element_type=jnp.float32)
    td = td + dec_b3_ref[...]
    lc_ref[...] = lc
    te_ref[...] = te
    td_ref[...] = td
    c2_ref[...] = jnp.sum(lc * lc, axis=1, keepdims=True)


def _tokens_body(x_ref, lct_ref, c2_ref, tok_ref):
    x = x_ref[...]
    m = jnp.dot(x, lct_ref[...], preferred_element_type=jnp.float32)
    d = (jnp.sum(x * x, axis=1, keepdims=True) + c2_ref[...]) - 2.0 * m
    v = d.shape[1]
    dmin = jnp.min(d, axis=1, keepdims=True)
    ids = lax.broadcasted_iota(jnp.int32, d.shape, 1)
    tok_ref[0, 0, :] = jnp.min(jnp.where(d <= dmin, ids, jnp.int32(v)), axis=1)


def _encode_body(xa_ref, enca_ref, z_ref):
    z_ref[...] = jnp.dot(xa_ref[...], enca_ref[...],
                         preferred_element_type=jnp.float32)


def _make_sc_gather(n_tok, v, e):
    nc, ns = 2, 16
    nw = nc * ns
    tpw = n_tok // nw
    ch = 128
    nch = tpw // ch
    nbuf = 2

    mesh = plsc.VectorSubcoreMesh(core_axis_name="c", subcore_axis_name="s",
                                  num_cores=nc, num_subcores=ns)

    @functools.partial(
        pl.kernel,
        out_type=(jax.ShapeDtypeStruct((n_tok, e), jnp.float32),
                  jax.ShapeDtypeStruct((n_tok * 3,), jnp.float32)),
        mesh=mesh,
        scratch_types=[
            pltpu.VMEM((tpw,), jnp.int32),
            [pltpu.VMEM((ch, e), jnp.float32) for _ in range(nbuf)],
            pltpu.VMEM((v * 3,), jnp.float32),
            pltpu.VMEM((tpw * 3,), jnp.float32),
            [pltpu.SemaphoreType.DMA for _ in range(nbuf)],
        ],
        compiler_params=pltpu.CompilerParams(needs_layout_passes=False),
    )
    def sc_gather(tok_hbm, te_hbm, td_hbm, zq_hbm, rec_hbm,
                  idx_v, gbufs, tdv, recv, gsems):
        w = lax.axis_index("s") * nc + lax.axis_index("c")
        base = w * tpw
        pltpu.sync_copy(tok_hbm.at[pl.ds(base, tpw)], idx_v)
        pltpu.sync_copy(td_hbm, tdv)

        gh = [None] * nbuf
        gh[0] = pltpu.async_copy(
            te_hbm.at[idx_v.at[pl.ds(0, ch)]], gbufs[0], gsems[0])
        for k in range(nch):
            b = k % nbuf
            if k + 1 < nch:
                bn = (k + 1) % nbuf
                gh[bn] = pltpu.async_copy(
                    te_hbm.at[idx_v.at[pl.ds((k + 1) * ch, ch)]],
                    gbufs[bn], gsems[bn])
            gh[b].wait()
            pltpu.sync_copy(gbufs[b], zq_hbm.at[pl.ds(base + k * ch, ch)])

        def rec_group(g, carry):
            idx = idx_v[pl.ds(g * 16, 16)]
            f = idx * 3
            p = (g * 16 + lax.broadcasted_iota(jnp.int32, (16,), 0)) * 3
            plsc.store_scatter(recv, [p], plsc.load_gather(tdv, [f]))
            plsc.store_scatter(recv, [p + 1], plsc.load_gather(tdv, [f + 1]))
            plsc.store_scatter(recv, [p + 2], plsc.load_gather(tdv, [f + 2]))
            return carry

        lax.fori_loop(0, tpw // 16, rec_group, 0)
        pltpu.sync_copy(recv, rec_hbm.at[pl.ds(w * tpw * 3, tpw * 3)])

    return sc_gather


def kernel(x, codebook_w, enc_w, enc_b, cm_w1, cm_b1, cm_g1, cm_be1,
           cm_w2, cm_b2, cm_g2, cm_be2, cm_w3, cm_b3,
           dec_w1, dec_b1, dec_w2, dec_b2, dec_w3, dec_b3):
    b, in_dim = x.shape
    v, d = codebook_w.shape
    e = enc_w.shape[1]
    k_tok = in_dim // d
    n = b * k_tok

    x_flat = x.reshape(n, d)
    row = lambda a: a.reshape(1, -1)

    lc, te, td, c2 = pl.pallas_call(
        _tables_body,
        out_shape=(jax.ShapeDtypeStruct((v, d), jnp.float32),
                   jax.ShapeDtypeStruct((v, e), jnp.float32),
                   jax.ShapeDtypeStruct((v, d), jnp.float32),
                   jax.ShapeDtypeStruct((v, 1), jnp.float32)),
    )(codebook_w, cm_w1, row(cm_b1), row(cm_g1), row(cm_be1),
      cm_w2, row(cm_b2), row(cm_g2), row(cm_be2),
      cm_w3, row(cm_b3), enc_w, row(enc_b),
      dec_w1, row(dec_b1), dec_w2, row(dec_b2), dec_w3, row(dec_b3))

    ones = jnp.ones((n, 1), jnp.float32)
    x_aug = jnp.concatenate([x_flat, ones], axis=1)
    enc_a = jnp.concatenate([enc_w, row(enc_b)], axis=0)
    lct = lc.T
    c2r = c2.reshape(1, v)

    grid = 32
    t = n // grid
    tok3 = pl.pallas_call(
        _tokens_body,
        grid=(grid,),
        in_specs=[pl.BlockSpec((t, d), lambda i: (i, 0)),
                  pl.BlockSpec((d, v), lambda i: (0, 0)),
                  pl.BlockSpec((1, v), lambda i: (0, 0))],
        out_specs=pl.BlockSpec((1, 1, t), lambda i: (i, 0, 0)),
        out_shape=jax.ShapeDtypeStruct((grid, 1, t), jnp.int32),
    )(x_flat, lct, c2r)

    tokens = tok3.reshape(n)
    zq, rec_flat = _make_sc_gather(n, v, e)(tokens, te, td.reshape(v * 3))

    z = pl.pallas_call(
        _encode_body,
        grid=(grid,),
        in_specs=[pl.BlockSpec((t, d + 1), lambda i: (i, 0)),
                  pl.BlockSpec((d + 1, e), lambda i: (0, 0))],
        out_specs=pl.BlockSpec((t, e), lambda i: (i, 0)),
        out_shape=jax.ShapeDtypeStruct((n, e), jnp.float32),
    )(x_aug, enc_a)

    return (z.reshape(b, k_tok, e),
            zq.reshape(b, k_tok, e),
            rec_flat.reshape(b, in_dim))


# speedup vs baseline: 1.0256x; 1.0256x over previous
import functools

import jax
import jax.numpy as jnp
from jax import lax
from jax.experimental import pallas as pl
from jax.experimental.pallas import tpu as pltpu
from jax.experimental.pallas import tpu_sc as plsc


def _ln(h, g, b):
    m = jnp.mean(h, axis=-1, keepdims=True)
    v = jnp.var(h, axis=-1, keepdims=True)
    return (h - m) / jnp.sqrt(v + 1e-5) * g + b


def _silu(h):
    return h * jax.nn.sigmoid(h)


def _tables_body(cb_ref, cm_w1_ref, cm_b1_ref, cm_g1_ref, cm_be1_ref,
                 cm_w2_ref, cm_b2_ref, cm_g2_ref, cm_be2_ref,
                 cm_w3_ref, cm_b3_ref, enc_w_ref, enc_b_ref,
                 dec_w1_ref, dec_b1_ref, dec_w2_ref, dec_b2_ref,
                 dec_w3_ref, dec_b3_ref,
                 lc_ref, te_ref, td_ref, c2_ref):
    cb = cb_ref[...]
    h = jnp.dot(cb, cm_w1_ref[...], preferred_element_type=jnp.float32)
    h = _silu(_ln(h + cm_b1_ref[...], cm_g1_ref[...], cm_be1_ref[...]))
    h = jnp.dot(h, cm_w2_ref[...], preferred_element_type=jnp.float32)
    h = _silu(_ln(h + cm_b2_ref[...], cm_g2_ref[...], cm_be2_ref[...]))
    lc = jnp.dot(h, cm_w3_ref[...], preferred_element_type=jnp.float32)
    lc = lc + cm_b3_ref[...]
    te = jnp.dot(lc, enc_w_ref[...], preferred_element_type=jnp.float32)
    te = te + enc_b_ref[...]
    hd = _silu(jnp.dot(te, dec_w1_ref[...], preferred_element_type=jnp.float32)
               + dec_b1_ref[...])
    hd = _silu(jnp.dot(hd, dec_w2_ref[...], preferred_element_type=jnp.float32)
               + dec_b2_ref[...])
    td = jnp.dot(hd, dec_w3_ref[...], preferred_element_type=jnp.float32)
    td = td + dec_b3_ref[...]
    lc_ref[...] = lc
    te_ref[...] = te
    td_ref[...] = td
    c2_ref[...] = jnp.sum(lc * lc, axis=1, keepdims=True)


def _tokens_body(x_ref, lct_ref, c2_ref, enca_ref, xa_ref, z_ref, tok_ref):
    z_ref[...] = jnp.dot(xa_ref[...], enca_ref[...],
                         preferred_element_type=jnp.float32)
    x = x_ref[...]
    m = jnp.dot(x, lct_ref[...], preferred_element_type=jnp.float32)
    d = (jnp.sum(x * x, axis=1, keepdims=True) + c2_ref[...]) - 2.0 * m
    v = d.shape[1]
    dmin = jnp.min(d, axis=1, keepdims=True)
    ids = lax.broadcasted_iota(jnp.int32, d.shape, 1)
    tok_ref[0, 0, :] = jnp.min(jnp.where(d <= dmin, ids, jnp.int32(v)), axis=1)


def _encode_body(xa_ref, enca_ref, z_ref):
    z_ref[...] = jnp.dot(xa_ref[...], enca_ref[...],
                         preferred_element_type=jnp.float32)


def _make_sc_gather(n_tok, v, e):
    nc, ns = 2, 16
    nw = nc * ns
    tpw = n_tok // nw
    ch = 128
    nch = tpw // ch
    nbuf = 2

    mesh = plsc.VectorSubcoreMesh(core_axis_name="c", subcore_axis_name="s",
                                  num_cores=nc, num_subcores=ns)

    @functools.partial(
        pl.kernel,
        out_type=(jax.ShapeDtypeStruct((n_tok, e), jnp.float32),
                  jax.ShapeDtypeStruct((n_tok * 3,), jnp.float32)),
        mesh=mesh,
        scratch_types=[
            pltpu.VMEM((tpw,), jnp.int32),
            [pltpu.VMEM((ch, e), jnp.float32) for _ in range(nbuf)],
            pltpu.VMEM((v * 3,), jnp.float32),
            pltpu.VMEM((tpw * 3,), jnp.float32),
            [pltpu.SemaphoreType.DMA for _ in range(nbuf)],
        ],
        compiler_params=pltpu.CompilerParams(needs_layout_passes=False),
    )
    def sc_gather(tok_hbm, te_hbm, td_hbm, zq_hbm, rec_hbm,
                  idx_v, gbufs, tdv, recv, gsems):
        w = lax.axis_index("s") * nc + lax.axis_index("c")
        base = w * tpw
        pltpu.sync_copy(tok_hbm.at[pl.ds(base, tpw)], idx_v)
        pltpu.sync_copy(td_hbm, tdv)

        gh = [None] * nbuf
        gh[0] = pltpu.async_copy(
            te_hbm.at[idx_v.at[pl.ds(0, ch)]], gbufs[0], gsems[0])
        for k in range(nch):
            b = k % nbuf
            if k + 1 < nch:
                bn = (k + 1) % nbuf
                gh[bn] = pltpu.async_copy(
                    te_hbm.at[idx_v.at[pl.ds((k + 1) * ch, ch)]],
                    gbufs[bn], gsems[bn])
            gh[b].wait()
            pltpu.sync_copy(gbufs[b], zq_hbm.at[pl.ds(base + k * ch, ch)])

        def rec_group(g, carry):
            idx = idx_v[pl.ds(g * 16, 16)]
            f = idx * 3
            p = (g * 16 + lax.broadcasted_iota(jnp.int32, (16,), 0)) * 3
            plsc.store_scatter(recv, [p], plsc.load_gather(tdv, [f]))
            plsc.store_scatter(recv, [p + 1], plsc.load_gather(tdv, [f + 1]))
            plsc.store_scatter(recv, [p + 2], plsc.load_gather(tdv, [f + 2]))
            return carry

        lax.fori_loop(0, tpw // 16, rec_group, 0)
        pltpu.sync_copy(recv, rec_hbm.at[pl.ds(w * tpw * 3, tpw * 3)])

    return sc_gather


def kernel(x, codebook_w, enc_w, enc_b, cm_w1, cm_b1, cm_g1, cm_be1,
           cm_w2, cm_b2, cm_g2, cm_be2, cm_w3, cm_b3,
           dec_w1, dec_b1, dec_w2, dec_b2, dec_w3, dec_b3):
    b, in_dim = x.shape
    v, d = codebook_w.shape
    e = enc_w.shape[1]
    k_tok = in_dim // d
    n = b * k_tok

    x_flat = x.reshape(n, d)
    row = lambda a: a.reshape(1, -1)

    lc, te, td, c2 = pl.pallas_call(
        _tables_body,
        out_shape=(jax.ShapeDtypeStruct((v, d), jnp.float32),
                   jax.ShapeDtypeStruct((v, e), jnp.float32),
                   jax.ShapeDtypeStruct((v, d), jnp.float32),
                   jax.ShapeDtypeStruct((v, 1), jnp.float32)),
    )(codebook_w, cm_w1, row(cm_b1), row(cm_g1), row(cm_be1),
      cm_w2, row(cm_b2), row(cm_g2), row(cm_be2),
      cm_w3, row(cm_b3), enc_w, row(enc_b),
      dec_w1, row(dec_b1), dec_w2, row(dec_b2), dec_w3, row(dec_b3))

    ones = jnp.ones((n, 1), jnp.float32)
    x_aug = jnp.concatenate([x_flat, ones], axis=1)
    enc_a = jnp.concatenate([enc_w, row(enc_b)], axis=0)
    lct = lc.T
    c2r = c2.reshape(1, v)

    grid = 32
    t = n // grid
    z, tok3 = pl.pallas_call(
        _tokens_body,
        grid=(grid,),
        in_specs=[pl.BlockSpec((t, d), lambda i: (i, 0)),
                  pl.BlockSpec((d, v), lambda i: (0, 0)),
                  pl.BlockSpec((1, v), lambda i: (0, 0)),
                  pl.BlockSpec((d + 1, e), lambda i: (0, 0)),
                  pl.BlockSpec((t, d + 1), lambda i: (i, 0))],
        out_specs=(pl.BlockSpec((t, e), lambda i: (i, 0)),
                   pl.BlockSpec((1, 1, t), lambda i: (i, 0, 0))),
        out_shape=(jax.ShapeDtypeStruct((n, e), jnp.float32),
                   jax.ShapeDtypeStruct((grid, 1, t), jnp.int32)),
    )(x_flat, lct, c2r, enc_a, x_aug)

    tokens = tok3.reshape(n)
    zq, rec_flat = _make_sc_gather(n, v, e)(tokens, te, td.reshape(v * 3))

    return (z.reshape(b, k_tok, e),
            zq.reshape(b, k_tok, e),
            rec_flat.reshape(b, in_dim))


# speedup vs baseline: 1.1062x; 1.0786x over previous
import functools

import jax
import jax.numpy as jnp
from jax import lax
from jax.experimental import pallas as pl
from jax.experimental.pallas import tpu as pltpu
from jax.experimental.pallas import tpu_sc as plsc


def _ln(h, g, b):
    m = jnp.mean(h, axis=-1, keepdims=True)
    v = jnp.var(h, axis=-1, keepdims=True)
    return (h - m) / jnp.sqrt(v + 1e-5) * g + b


def _silu(h):
    return h * jax.nn.sigmoid(h)


def _tables_body(cb_ref, cm_w1_ref, cm_b1_ref, cm_g1_ref, cm_be1_ref,
                 cm_w2_ref, cm_b2_ref, cm_g2_ref, cm_be2_ref,
                 cm_w3_ref, cm_b3_ref, enc_w_ref, enc_b_ref,
                 dec_w1_ref, dec_b1_ref, dec_w2_ref, dec_b2_ref,
                 dec_w3_ref, dec_b3_ref,
                 lc_ref, te_ref, td_ref, c2_ref):
    cb = cb_ref[...]
    h = jnp.dot(cb, cm_w1_ref[...], preferred_element_type=jnp.float32)
    h = _silu(_ln(h + cm_b1_ref[...], cm_g1_ref[...], cm_be1_ref[...]))
    h = jnp.dot(h, cm_w2_ref[...], preferred_element_type=jnp.float32)
    h = _silu(_ln(h + cm_b2_ref[...], cm_g2_ref[...], cm_be2_ref[...]))
    lc = jnp.dot(h, cm_w3_ref[...], preferred_element_type=jnp.float32)
    lc = lc + cm_b3_ref[...]
    te = jnp.dot(lc, enc_w_ref[...], preferred_element_type=jnp.float32)
    te = te + enc_b_ref[...]
    hd = _silu(jnp.dot(te, dec_w1_ref[...], preferred_element_type=jnp.float32)
               + dec_b1_ref[...])
    hd = _silu(jnp.dot(hd, dec_w2_ref[...], preferred_element_type=jnp.float32)
               + dec_b2_ref[...])
    td = jnp.dot(hd, dec_w3_ref[...], preferred_element_type=jnp.float32)
    td = td + dec_b3_ref[...]
    lc_ref[...] = lc
    te_ref[...] = te
    td_ref[...] = td
    c2_ref[...] = jnp.sum(lc * lc, axis=1, keepdims=True)


def _tokens_body(x_ref, lct_ref, c2_ref, enc_w_ref, enc_b_ref, z_ref, tok_ref):
    z_ref[...] = (jnp.dot(x_ref[...], enc_w_ref[...],
                          preferred_element_type=jnp.float32)
                  + enc_b_ref[...])
    x = x_ref[...]
    m = jnp.dot(x, lct_ref[...], preferred_element_type=jnp.float32)
    d = (jnp.sum(x * x, axis=1, keepdims=True) + c2_ref[...]) - 2.0 * m
    v = d.shape[1]
    dmin = jnp.min(d, axis=1, keepdims=True)
    ids = lax.broadcasted_iota(jnp.int32, d.shape, 1)
    tok_ref[0, 0, :] = jnp.min(jnp.where(d <= dmin, ids, jnp.int32(v)), axis=1)


def _make_sc_gather(n_tok, v, e):
    nc, ns = 2, 16
    nw = nc * ns
    tpw = n_tok // nw
    ch = 128
    nch = tpw // ch
    nbuf = 2

    mesh = plsc.VectorSubcoreMesh(core_axis_name="c", subcore_axis_name="s",
                                  num_cores=nc, num_subcores=ns)

    @functools.partial(
        pl.kernel,
        out_type=(jax.ShapeDtypeStruct((n_tok, e), jnp.float32),
                  jax.ShapeDtypeStruct((n_tok * 3,), jnp.float32)),
        mesh=mesh,
        scratch_types=[
            pltpu.VMEM((tpw,), jnp.int32),
            [pltpu.VMEM((ch, e), jnp.float32) for _ in range(nbuf)],
            pltpu.VMEM((v * 3,), jnp.float32),
            pltpu.VMEM((tpw * 3,), jnp.float32),
            [pltpu.SemaphoreType.DMA for _ in range(nbuf)],
        ],
        compiler_params=pltpu.CompilerParams(needs_layout_passes=False),
    )
    def sc_gather(tok_hbm, te_hbm, td_hbm, zq_hbm, rec_hbm,
                  idx_v, gbufs, tdv, recv, gsems):
        w = lax.axis_index("s") * nc + lax.axis_index("c")
        base = w * tpw
        pltpu.sync_copy(tok_hbm.at[pl.ds(base, tpw)], idx_v)
        pltpu.sync_copy(td_hbm, tdv)

        gh = [None] * nbuf
        gh[0] = pltpu.async_copy(
            te_hbm.at[idx_v.at[pl.ds(0, ch)]], gbufs[0], gsems[0])
        for k in range(nch):
            b = k % nbuf
            if k + 1 < nch:
                bn = (k + 1) % nbuf
                gh[bn] = pltpu.async_copy(
                    te_hbm.at[idx_v.at[pl.ds((k + 1) * ch, ch)]],
                    gbufs[bn], gsems[bn])
            gh[b].wait()
            pltpu.sync_copy(gbufs[b], zq_hbm.at[pl.ds(base + k * ch, ch)])

        def rec_group(g, carry):
            idx = idx_v[pl.ds(g * 16, 16)]
            f = idx * 3
            p = (g * 16 + lax.broadcasted_iota(jnp.int32, (16,), 0)) * 3
            plsc.store_scatter(recv, [p], plsc.load_gather(tdv, [f]))
            plsc.store_scatter(recv, [p + 1], plsc.load_gather(tdv, [f + 1]))
            plsc.store_scatter(recv, [p + 2], plsc.load_gather(tdv, [f + 2]))
            return carry

        lax.fori_loop(0, tpw // 16, rec_group, 0)
        pltpu.sync_copy(recv, rec_hbm.at[pl.ds(w * tpw * 3, tpw * 3)])

    return sc_gather


def kernel(x, codebook_w, enc_w, enc_b, cm_w1, cm_b1, cm_g1, cm_be1,
           cm_w2, cm_b2, cm_g2, cm_be2, cm_w3, cm_b3,
           dec_w1, dec_b1, dec_w2, dec_b2, dec_w3, dec_b3):
    b, in_dim = x.shape
    v, d = codebook_w.shape
    e = enc_w.shape[1]
    k_tok = in_dim // d
    n = b * k_tok

    x_flat = x.reshape(n, d)
    row = lambda a: a.reshape(1, -1)

    lc, te, td, c2 = pl.pallas_call(
        _tables_body,
        out_shape=(jax.ShapeDtypeStruct((v, d), jnp.float32),
                   jax.ShapeDtypeStruct((v, e), jnp.float32),
                   jax.ShapeDtypeStruct((v, d), jnp.float32),
                   jax.ShapeDtypeStruct((v, 1), jnp.float32)),
    )(codebook_w, cm_w1, row(cm_b1), row(cm_g1), row(cm_be1),
      cm_w2, row(cm_b2), row(cm_g2), row(cm_be2),
      cm_w3, row(cm_b3), enc_w, row(enc_b),
      dec_w1, row(dec_b1), dec_w2, row(dec_b2), dec_w3, row(dec_b3))

    lct = lc.T
    c2r = c2.reshape(1, v)

    grid = 32
    t = n // grid
    z, tok3 = pl.pallas_call(
        _tokens_body,
        grid=(grid,),
        in_specs=[pl.BlockSpec((t, d), lambda i: (i, 0)),
                  pl.BlockSpec((d, v), lambda i: (0, 0)),
                  pl.BlockSpec((1, v), lambda i: (0, 0)),
                  pl.BlockSpec((d, e), lambda i: (0, 0)),
                  pl.BlockSpec((1, e), lambda i: (0, 0))],
        out_specs=(pl.BlockSpec((t, e), lambda i: (i, 0)),
                   pl.BlockSpec((1, 1, t), lambda i: (i, 0, 0))),
        out_shape=(jax.ShapeDtypeStruct((n, e), jnp.float32),
                   jax.ShapeDtypeStruct((grid, 1, t), jnp.int32)),
    )(x_flat, lct, c2r, enc_w, row(enc_b))

    tokens = tok3.reshape(n)
    zq, rec_flat = _make_sc_gather(n, v, e)(tokens, te, td.reshape(v * 3))

    return (z.reshape(b, k_tok, e),
            zq.reshape(b, k_tok, e),
            rec_flat.reshape(b, in_dim))


# speedup vs baseline: 1.1310x; 1.0224x over previous
import functools

import jax
import jax.numpy as jnp
from jax import lax
from jax.experimental import pallas as pl
from jax.experimental.pallas import tpu as pltpu
from jax.experimental.pallas import tpu_sc as plsc


def _ln(h, g, b):
    m = jnp.mean(h, axis=-1, keepdims=True)
    v = jnp.var(h, axis=-1, keepdims=True)
    return (h - m) / jnp.sqrt(v + 1e-5) * g + b


def _silu(h):
    return h * jax.nn.sigmoid(h)


def _dot(a, b, dims):
    return lax.dot_general(a, b, (dims, ((), ())),
                           preferred_element_type=jnp.float32)


def _main_body(x_ref, cb_ref, cm_w1_ref, cm_b1_ref, cm_g1_ref, cm_be1_ref,
               cm_w2_ref, cm_b2_ref, cm_g2_ref, cm_be2_ref,
               cm_w3_ref, cm_b3c_ref, enc_w_ref, enc_b_ref,
               dec_w1_ref, dec_b1_ref, dec_w2_ref, dec_b2_ref,
               dec_w3_ref, dec_b3_ref,
               z_ref, tok_ref, te_ref, td_ref,
               lct_s, c2_s):
    i = pl.program_id(0)

    @pl.when(i == 0)
    def _tables():
        cb = cb_ref[...]
        h = _dot(cb, cm_w1_ref[...], ((1,), (0,)))
        h = _silu(_ln(h + cm_b1_ref[...], cm_g1_ref[...], cm_be1_ref[...]))
        h = _dot(h, cm_w2_ref[...], ((1,), (0,)))
        h = _silu(_ln(h + cm_b2_ref[...], cm_g2_ref[...], cm_be2_ref[...]))
        lct = _dot(cm_w3_ref[...], h, ((0,), (1,))) + cm_b3c_ref[...]
        lct_s[...] = lct
        c2_s[...] = jnp.sum(lct * lct, axis=0, keepdims=True)
        te = _dot(lct, enc_w_ref[...], ((0,), (0,))) + enc_b_ref[...]
        te_ref[...] = te
        hd = _silu(_dot(te, dec_w1_ref[...], ((1,), (0,))) + dec_b1_ref[...])
        hd = _silu(_dot(hd, dec_w2_ref[...], ((1,), (0,))) + dec_b2_ref[...])
        td_ref[...] = _dot(hd, dec_w3_ref[...], ((1,), (0,))) + dec_b3_ref[...]

    x = x_ref[...]
    z_ref[...] = _dot(x, enc_w_ref[...], ((1,), (0,))) + enc_b_ref[...]
    m = _dot(x, lct_s[...], ((1,), (0,)))
    d = (jnp.sum(x * x, axis=1, keepdims=True) + c2_s[...]) - 2.0 * m
    v = d.shape[1]
    dmin = jnp.min(d, axis=1, keepdims=True)
    ids = lax.broadcasted_iota(jnp.int32, d.shape, 1)
    tok_ref[0, 0, :] = jnp.min(jnp.where(d <= dmin, ids, jnp.int32(v)), axis=1)


def _make_sc_gather(n_tok, v, e):
    nc, ns = 2, 16
    nw = nc * ns
    tpw = n_tok // nw
    ch = 128
    nch = tpw // ch
    nbuf = 2

    mesh = plsc.VectorSubcoreMesh(core_axis_name="c", subcore_axis_name="s",
                                  num_cores=nc, num_subcores=ns)

    @functools.partial(
        pl.kernel,
        out_type=(jax.ShapeDtypeStruct((n_tok, e), jnp.float32),
                  jax.ShapeDtypeStruct((n_tok * 3,), jnp.float32)),
        mesh=mesh,
        scratch_types=[
            pltpu.VMEM((tpw,), jnp.int32),
            [pltpu.VMEM((ch, e), jnp.float32) for _ in range(nbuf)],
            pltpu.VMEM((v * 3,), jnp.float32),
            pltpu.VMEM((tpw * 3,), jnp.float32),
            [pltpu.SemaphoreType.DMA for _ in range(nbuf)],
        ],
        compiler_params=pltpu.CompilerParams(needs_layout_passes=False),
    )
    def sc_gather(tok_hbm, te_hbm, td_hbm, zq_hbm, rec_hbm,
                  idx_v, gbufs, tdv, recv, gsems):
        w = lax.axis_index("s") * nc + lax.axis_index("c")
        base = w * tpw
        pltpu.sync_copy(tok_hbm.at[pl.ds(base, tpw)], idx_v)
        pltpu.sync_copy(td_hbm, tdv)

        gh = [None] * nbuf
        gh[0] = pltpu.async_copy(
            te_hbm.at[idx_v.at[pl.ds(0, ch)]], gbufs[0], gsems[0])
        for k in range(nch):
            b = k % nbuf
            if k + 1 < nch:
                bn = (k + 1) % nbuf
                gh[bn] = pltpu.async_copy(
                    te_hbm.at[idx_v.at[pl.ds((k + 1) * ch, ch)]],
                    gbufs[bn], gsems[bn])
            gh[b].wait()
            pltpu.sync_copy(gbufs[b], zq_hbm.at[pl.ds(base + k * ch, ch)])

        def rec_group(g, carry):
            idx = idx_v[pl.ds(g * 16, 16)]
            f = idx * 3
            p = (g * 16 + lax.broadcasted_iota(jnp.int32, (16,), 0)) * 3
            plsc.store_scatter(recv, [p], plsc.load_gather(tdv, [f]))
            plsc.store_scatter(recv, [p + 1], plsc.load_gather(tdv, [f + 1]))
            plsc.store_scatter(recv, [p + 2], plsc.load_gather(tdv, [f + 2]))
            return carry

        lax.fori_loop(0, tpw // 16, rec_group, 0)
        pltpu.sync_copy(recv, rec_hbm.at[pl.ds(w * tpw * 3, tpw * 3)])

    return sc_gather


def kernel(x, codebook_w, enc_w, enc_b, cm_w1, cm_b1, cm_g1, cm_be1,
           cm_w2, cm_b2, cm_g2, cm_be2, cm_w3, cm_b3,
           dec_w1, dec_b1, dec_w2, dec_b2, dec_w3, dec_b3):
    b, in_dim = x.shape
    v, d = codebook_w.shape
    e = enc_w.shape[1]
    e2 = dec_w1.shape[1]
    k_tok = in_dim // d
    n = b * k_tok

    x_flat = x.reshape(n, d)
    row = lambda a: a.reshape(1, -1)
    full = lambda shape: pl.BlockSpec(shape, lambda i: tuple(0 for _ in shape))

    grid = 32
    t = n // grid
    z, tok3, te, td = pl.pallas_call(
        _main_body,
        grid=(grid,),
        in_specs=[pl.BlockSpec((t, d), lambda i: (i, 0)),
                  full((v, d)),
                  full((d, e)), full((1, e)), full((1, e)), full((1, e)),
                  full((e, e)), full((1, e)), full((1, e)), full((1, e)),
                  full((e, d)), full((d, 1)),
                  full((d, e)), full((1, e)),
                  full((e, e2)), full((1, e2)),
                  full((e2, e2)), full((1, e2)),
                  full((e2, d)), full((1, d))],
        out_specs=(pl.BlockSpec((t, e), lambda i: (i, 0)),
                   pl.BlockSpec((1, 1, t), lambda i: (i, 0, 0)),
                   full((v, e)),
                   full((v, d))),
        out_shape=(jax.ShapeDtypeStruct((n, e), jnp.float32),
                   jax.ShapeDtypeStruct((grid, 1, t), jnp.int32),
                   jax.ShapeDtypeStruct((v, e), jnp.float32),
                   jax.ShapeDtypeStruct((v, d), jnp.float32)),
        scratch_shapes=[pltpu.VMEM((d, v), jnp.float32),
                        pltpu.VMEM((1, v), jnp.float32)],
    )(x_flat, codebook_w,
      cm_w1, row(cm_b1), row(cm_g1), row(cm_be1),
      cm_w2, row(cm_b2), row(cm_g2), row(cm_be2),
      cm_w3, cm_b3.reshape(-1, 1),
      enc_w, row(enc_b),
      dec_w1, row(dec_b1), dec_w2, row(dec_b2), dec_w3, row(dec_b3))

    tokens = tok3.reshape(n)
    zq, rec_flat = _make_sc_gather(n, v, e)(tokens, te, td.reshape(v * 3))

    return (z.reshape(b, k_tok, e),
            zq.reshape(b, k_tok, e),
            rec_flat.reshape(b, in_dim))


# speedup vs baseline: 1.1424x; 1.0100x over previous
import functools

import jax
import jax.numpy as jnp
from jax import lax
from jax.experimental import pallas as pl
from jax.experimental.pallas import tpu as pltpu
from jax.experimental.pallas import tpu_sc as plsc


def _ln(h, g, b):
    m = jnp.mean(h, axis=-1, keepdims=True)
    v = jnp.var(h, axis=-1, keepdims=True)
    return (h - m) / jnp.sqrt(v + 1e-5) * g + b


def _silu(h):
    return h * jax.nn.sigmoid(h)


def _dot(a, b, dims):
    return lax.dot_general(a, b, (dims, ((), ())),
                           preferred_element_type=jnp.float32)


def _main_body(x_ref, cb_ref, cm_w1_ref, cm_b1_ref, cm_g1_ref, cm_be1_ref,
               cm_w2_ref, cm_b2_ref, cm_g2_ref, cm_be2_ref,
               cm_w3_ref, cm_b3c_ref, enc_w_ref, enc_b_ref,
               dec_w1_ref, dec_b1_ref, dec_w2_ref, dec_b2_ref,
               dec_w3_ref, dec_b3_ref,
               z_ref, tok_ref, te_ref, td_ref,
               lct_s, c2_s):
    i = pl.program_id(0)

    @pl.when(i == 0)
    def _tables():
        cb = cb_ref[...]
        h = _dot(cb, cm_w1_ref[...], ((1,), (0,)))
        h = _silu(_ln(h + cm_b1_ref[...], cm_g1_ref[...], cm_be1_ref[...]))
        h = _dot(h, cm_w2_ref[...], ((1,), (0,)))
        h = _silu(_ln(h + cm_b2_ref[...], cm_g2_ref[...], cm_be2_ref[...]))
        lct = _dot(cm_w3_ref[...], h, ((0,), (1,))) + cm_b3c_ref[...]
        lct_s[...] = lct
        c2_s[...] = jnp.sum(lct * lct, axis=0, keepdims=True)
        te = _dot(lct, enc_w_ref[...], ((0,), (0,))) + enc_b_ref[...]
        te_ref[...] = te
        hd = _silu(_dot(te, dec_w1_ref[...], ((1,), (0,))) + dec_b1_ref[...])
        hd = _silu(_dot(hd, dec_w2_ref[...], ((1,), (0,))) + dec_b2_ref[...])
        td_ref[...] = _dot(hd, dec_w3_ref[...], ((1,), (0,))) + dec_b3_ref[...]

    x = x_ref[...]
    z_ref[...] = _dot(x, enc_w_ref[...], ((1,), (0,))) + enc_b_ref[...]
    m = _dot(x, lct_s[...], ((1,), (0,)))
    d = c2_s[...] - 2.0 * m
    v = d.shape[1]
    dmin = jnp.min(d, axis=1, keepdims=True)
    ids = lax.broadcasted_iota(jnp.int32, d.shape, 1)
    tok_ref[0, 0, :] = jnp.min(jnp.where(d <= dmin, ids, jnp.int32(v)), axis=1)


def _make_sc_gather(n_tok, v, e):
    nc, ns = 2, 16
    nw = nc * ns
    tpw = n_tok // nw
    ch = 128
    nch = tpw // ch
    nbuf = 2

    mesh = plsc.VectorSubcoreMesh(core_axis_name="c", subcore_axis_name="s",
                                  num_cores=nc, num_subcores=ns)

    @functools.partial(
        pl.kernel,
        out_type=(jax.ShapeDtypeStruct((n_tok, e), jnp.float32),
                  jax.ShapeDtypeStruct((n_tok * 3,), jnp.float32)),
        mesh=mesh,
        scratch_types=[
            pltpu.VMEM((tpw,), jnp.int32),
            [pltpu.VMEM((ch, e), jnp.float32) for _ in range(nbuf)],
            pltpu.VMEM((v * 3,), jnp.float32),
            pltpu.VMEM((tpw * 3,), jnp.float32),
            [pltpu.SemaphoreType.DMA for _ in range(nbuf)],
        ],
        compiler_params=pltpu.CompilerParams(needs_layout_passes=False),
    )
    def sc_gather(tok_hbm, te_hbm, td_hbm, zq_hbm, rec_hbm,
                  idx_v, gbufs, tdv, recv, gsems):
        w = lax.axis_index("s") * nc + lax.axis_index("c")
        base = w * tpw
        pltpu.sync_copy(tok_hbm.at[pl.ds(base, tpw)], idx_v)
        pltpu.sync_copy(td_hbm, tdv)

        gh = [None] * nbuf
        gh[0] = pltpu.async_copy(
            te_hbm.at[idx_v.at[pl.ds(0, ch)]], gbufs[0], gsems[0])
        for k in range(nch):
            b = k % nbuf
            if k + 1 < nch:
                bn = (k + 1) % nbuf
                gh[bn] = pltpu.async_copy(
                    te_hbm.at[idx_v.at[pl.ds((k + 1) * ch, ch)]],
                    gbufs[bn], gsems[bn])
            gh[b].wait()
            pltpu.sync_copy(gbufs[b], zq_hbm.at[pl.ds(base + k * ch, ch)])

        def rec_group(g, carry):
            idx = idx_v[pl.ds(g * 16, 16)]
            f = idx * 3
            p = (g * 16 + lax.broadcasted_iota(jnp.int32, (16,), 0)) * 3
            plsc.store_scatter(recv, [p], plsc.load_gather(tdv, [f]))
            plsc.store_scatter(recv, [p + 1], plsc.load_gather(tdv, [f + 1]))
            plsc.store_scatter(recv, [p + 2], plsc.load_gather(tdv, [f + 2]))
            return carry

        lax.fori_loop(0, tpw // 16, rec_group, 0)
        pltpu.sync_copy(recv, rec_hbm.at[pl.ds(w * tpw * 3, tpw * 3)])

    return sc_gather


def kernel(x, codebook_w, enc_w, enc_b, cm_w1, cm_b1, cm_g1, cm_be1,
           cm_w2, cm_b2, cm_g2, cm_be2, cm_w3, cm_b3,
           dec_w1, dec_b1, dec_w2, dec_b2, dec_w3, dec_b3):
    b, in_dim = x.shape
    v, d = codebook_w.shape
    e = enc_w.shape[1]
    e2 = dec_w1.shape[1]
    k_tok = in_dim // d
    n = b * k_tok

    x_flat = x.reshape(n, d)
    row = lambda a: a.reshape(1, -1)
    full = lambda shape: pl.BlockSpec(shape, lambda i: tuple(0 for _ in shape))

    grid = 32
    t = n // grid
    z, tok3, te, td = pl.pallas_call(
        _main_body,
        grid=(grid,),
        in_specs=[pl.BlockSpec((t, d), lambda i: (i, 0)),
                  full((v, d)),
                  full((d, e)), full((1, e)), full((1, e)), full((1, e)),
                  full((e, e)), full((1, e)), full((1, e)), full((1, e)),
                  full((e, d)), full((d, 1)),
                  full((d, e)), full((1, e)),
                  full((e, e2)), full((1, e2)),
                  full((e2, e2)), full((1, e2)),
                  full((e2, d)), full((1, d))],
        out_specs=(pl.BlockSpec((t, e), lambda i: (i, 0)),
                   pl.BlockSpec((1, 1, t), lambda i: (i, 0, 0)),
                   full((v, e)),
                   full((v, d))),
        out_shape=(jax.ShapeDtypeStruct((n, e), jnp.float32),
                   jax.ShapeDtypeStruct((grid, 1, t), jnp.int32),
                   jax.ShapeDtypeStruct((v, e), jnp.float32),
                   jax.ShapeDtypeStruct((v, d), jnp.float32)),
        scratch_shapes=[pltpu.VMEM((d, v), jnp.float32),
                        pltpu.VMEM((1, v), jnp.float32)],
    )(x_flat, codebook_w,
      cm_w1, row(cm_b1), row(cm_g1), row(cm_be1),
      cm_w2, row(cm_b2), row(cm_g2), row(cm_be2),
      cm_w3, cm_b3.reshape(-1, 1),
      enc_w, row(enc_b),
      dec_w1, row(dec_b1), dec_w2, row(dec_b2), dec_w3, row(dec_b3))

    tokens = tok3.reshape(n)
    zq, rec_flat = _make_sc_gather(n, v, e)(tokens, te, td.reshape(v * 3))

    return (z.reshape(b, k_tok, e),
            zq.reshape(b, k_tok, e),
            rec_flat.reshape(b, in_dim))


# speedup vs baseline: 1.1475x; 1.0045x over previous
import functools

import jax
import jax.numpy as jnp
from jax import lax
from jax.experimental import pallas as pl
from jax.experimental.pallas import tpu as pltpu
from jax.experimental.pallas import tpu_sc as plsc


def _ln(h, g, b):
    m = jnp.mean(h, axis=-1, keepdims=True)
    v = jnp.var(h, axis=-1, keepdims=True)
    return (h - m) / jnp.sqrt(v + 1e-5) * g + b


def _silu(h):
    return h * jax.nn.sigmoid(h)


def _dot(a, b, dims):
    return lax.dot_general(a, b, (dims, ((), ())),
                           preferred_element_type=jnp.float32)


def _main_body(x_ref, cb_ref, cm_w1_ref, cm_b1_ref, cm_g1_ref, cm_be1_ref,
               cm_w2_ref, cm_b2_ref, cm_g2_ref, cm_be2_ref,
               cm_w3_ref, cm_b3c_ref, enc_w_ref, enc_b_ref,
               dec_w1_ref, dec_b1_ref, dec_w2_ref, dec_b2_ref,
               dec_w3_ref, dec_b3_ref,
               z_ref, tok_ref, te_ref, td_ref,
               lct_s, c2_s):
    i = pl.program_id(0)

    @pl.when(i == 0)
    def _tables():
        cb = cb_ref[...]
        h = _dot(cb, cm_w1_ref[...], ((1,), (0,)))
        h = _silu(_ln(h + cm_b1_ref[...], cm_g1_ref[...], cm_be1_ref[...]))
        h = _dot(h, cm_w2_ref[...], ((1,), (0,)))
        h = _silu(_ln(h + cm_b2_ref[...], cm_g2_ref[...], cm_be2_ref[...]))
        lct = _dot(cm_w3_ref[...], h, ((0,), (1,))) + cm_b3c_ref[...]
        lct_s[...] = lct
        c2_s[...] = jnp.sum(lct * lct, axis=0, keepdims=True)
        te = _dot(lct, enc_w_ref[...], ((0,), (0,))) + enc_b_ref[...]
        te_ref[...] = te
        hd = _silu(_dot(te, dec_w1_ref[...], ((1,), (0,))) + dec_b1_ref[...])
        hd = _silu(_dot(hd, dec_w2_ref[...], ((1,), (0,))) + dec_b2_ref[...])
        td_ref[...] = _dot(hd, dec_w3_ref[...], ((1,), (0,))) + dec_b3_ref[...]

    x = x_ref[...]
    z_ref[...] = _dot(x, enc_w_ref[...], ((1,), (0,))) + enc_b_ref[...]
    m = _dot(x, lct_s[...], ((1,), (0,)))
    d = c2_s[...] - 2.0 * m
    v = d.shape[1]
    dmin = jnp.min(d, axis=1, keepdims=True)
    ids = lax.broadcasted_iota(jnp.int32, d.shape, 1)
    tok_ref[0, 0, :] = jnp.min(jnp.where(d <= dmin, ids, jnp.int32(v)), axis=1)


def _make_sc_gather(n_tok, v, e):
    nc, ns = 2, 16
    nw = nc * ns
    tpw = n_tok // nw
    ch = 128
    nch = tpw // ch
    nbuf = 3

    mesh = plsc.VectorSubcoreMesh(core_axis_name="c", subcore_axis_name="s",
                                  num_cores=nc, num_subcores=ns)

    @functools.partial(
        pl.kernel,
        out_type=(jax.ShapeDtypeStruct((n_tok, e), jnp.float32),
                  jax.ShapeDtypeStruct((n_tok * 3,), jnp.float32)),
        mesh=mesh,
        scratch_types=[
            pltpu.VMEM((tpw,), jnp.int32),
            [pltpu.VMEM((ch, e), jnp.float32) for _ in range(nbuf)],
            pltpu.VMEM((v * 3,), jnp.float32),
            pltpu.VMEM((tpw * 3,), jnp.float32),
            [pltpu.SemaphoreType.DMA for _ in range(nbuf)],
        ],
        compiler_params=pltpu.CompilerParams(needs_layout_passes=False),
    )
    def sc_gather(tok_hbm, te_hbm, td_hbm, zq_hbm, rec_hbm,
                  idx_v, gbufs, tdv, recv, gsems):
        w = lax.axis_index("s") * nc + lax.axis_index("c")
        base = w * tpw
        pltpu.sync_copy(tok_hbm.at[pl.ds(base, tpw)], idx_v)
        pltpu.sync_copy(td_hbm, tdv)

        depth = nbuf - 1
        gh = [None] * nbuf
        for k in range(depth):
            gh[k] = pltpu.async_copy(
                te_hbm.at[idx_v.at[pl.ds(k * ch, ch)]], gbufs[k], gsems[k])
        for k in range(nch):
            b = k % nbuf
            if k + depth < nch:
                bn = (k + depth) % nbuf
                gh[bn] = pltpu.async_copy(
                    te_hbm.at[idx_v.at[pl.ds((k + depth) * ch, ch)]],
                    gbufs[bn], gsems[bn])
            gh[b].wait()
            pltpu.sync_copy(gbufs[b], zq_hbm.at[pl.ds(base + k * ch, ch)])

        def rec_group(g, carry):
            idx = idx_v[pl.ds(g * 16, 16)]
            f = idx * 3
            p = (g * 16 + lax.broadcasted_iota(jnp.int32, (16,), 0)) * 3
            plsc.store_scatter(recv, [p], plsc.load_gather(tdv, [f]))
            plsc.store_scatter(recv, [p + 1], plsc.load_gather(tdv, [f + 1]))
            plsc.store_scatter(recv, [p + 2], plsc.load_gather(tdv, [f + 2]))
            return carry

        lax.fori_loop(0, tpw // 16, rec_group, 0)
        pltpu.sync_copy(recv, rec_hbm.at[pl.ds(w * tpw * 3, tpw * 3)])

    return sc_gather


def kernel(x, codebook_w, enc_w, enc_b, cm_w1, cm_b1, cm_g1, cm_be1,
           cm_w2, cm_b2, cm_g2, cm_be2, cm_w3, cm_b3,
           dec_w1, dec_b1, dec_w2, dec_b2, dec_w3, dec_b3):
    b, in_dim = x.shape
    v, d = codebook_w.shape
    e = enc_w.shape[1]
    e2 = dec_w1.shape[1]
    k_tok = in_dim // d
    n = b * k_tok

    x_flat = x.reshape(n, d)
    row = lambda a: a.reshape(1, -1)
    full = lambda shape: pl.BlockSpec(shape, lambda i: tuple(0 for _ in shape))

    grid = 32
    t = n // grid
    z, tok3, te, td = pl.pallas_call(
        _main_body,
        grid=(grid,),
        in_specs=[pl.BlockSpec((t, d), lambda i: (i, 0)),
                  full((v, d)),
                  full((d, e)), full((1, e)), full((1, e)), full((1, e)),
                  full((e, e)), full((1, e)), full((1, e)), full((1, e)),
                  full((e, d)), full((d, 1)),
                  full((d, e)), full((1, e)),
                  full((e, e2)), full((1, e2)),
                  full((e2, e2)), full((1, e2)),
                  full((e2, d)), full((1, d))],
        out_specs=(pl.BlockSpec((t, e), lambda i: (i, 0)),
                   pl.BlockSpec((1, 1, t), lambda i: (i, 0, 0)),
                   full((v, e)),
                   full((v, d))),
        out_shape=(jax.ShapeDtypeStruct((n, e), jnp.float32),
                   jax.ShapeDtypeStruct((grid, 1, t), jnp.int32),
                   jax.ShapeDtypeStruct((v, e), jnp.float32),
                   jax.ShapeDtypeStruct((v, d), jnp.float32)),
        scratch_shapes=[pltpu.VMEM((d, v), jnp.float32),
                        pltpu.VMEM((1, v), jnp.float32)],
    )(x_flat, codebook_w,
      cm_w1, row(cm_b1), row(cm_g1), row(cm_be1),
      cm_w2, row(cm_b2), row(cm_g2), row(cm_be2),
      cm_w3, cm_b3.reshape(-1, 1),
      enc_w, row(enc_b),
      dec_w1, row(dec_b1), dec_w2, row(dec_b2), dec_w3, row(dec_b3))

    tokens = tok3.reshape(n)
    zq, rec_flat = _make_sc_gather(n, v, e)(tokens, te, td.reshape(v * 3))

    return (z.reshape(b, k_tok, e),
            zq.reshape(b, k_tok, e),
            rec_flat.reshape(b, in_dim))


# speedup vs baseline: 1.1657x; 1.0158x over previous
import functools

import jax
import jax.numpy as jnp
from jax import lax
from jax.experimental import pallas as pl
from jax.experimental.pallas import tpu as pltpu
from jax.experimental.pallas import tpu_sc as plsc


def _ln(h, g, b):
    m = jnp.mean(h, axis=-1, keepdims=True)
    v = jnp.var(h, axis=-1, keepdims=True)
    return (h - m) / jnp.sqrt(v + 1e-5) * g + b


def _silu(h):
    return h * jax.nn.sigmoid(h)


def _dot(a, b, dims):
    return lax.dot_general(a, b, (dims, ((), ())),
                           preferred_element_type=jnp.float32)


def _main_body(x_ref, cb_ref, cm_w1_ref, cm_b1_ref, cm_g1_ref, cm_be1_ref,
               cm_w2_ref, cm_b2_ref, cm_g2_ref, cm_be2_ref,
               cm_w3_ref, cm_b3c_ref, enc_w_ref, enc_b_ref,
               dec_w1_ref, dec_b1_ref, dec_w2_ref, dec_b2_ref,
               dec_w3_ref, dec_b3_ref,
               z_ref, tok_ref, te_ref, td_ref,
               lct_s, c2_s):
    i = pl.program_id(0)

    @pl.when(i == 0)
    def _tables():
        cb = cb_ref[...]
        h = _dot(cb, cm_w1_ref[...], ((1,), (0,)))
        h = _silu(_ln(h + cm_b1_ref[...], cm_g1_ref[...], cm_be1_ref[...]))
        h = _dot(h, cm_w2_ref[...], ((1,), (0,)))
        h = _silu(_ln(h + cm_b2_ref[...], cm_g2_ref[...], cm_be2_ref[...]))
        lct = _dot(cm_w3_ref[...], h, ((0,), (1,))) + cm_b3c_ref[...]
        lct_s[...] = lct
        c2_s[...] = jnp.sum(lct * lct, axis=0, keepdims=True)
        te = _dot(lct, enc_w_ref[...], ((0,), (0,))) + enc_b_ref[...]
        te_ref[...] = te
        hd = _silu(_dot(te, dec_w1_ref[...], ((1,), (0,))) + dec_b1_ref[...])
        hd = _silu(_dot(hd, dec_w2_ref[...], ((1,), (0,))) + dec_b2_ref[...])
        td_ref[...] = _dot(hd, dec_w3_ref[...], ((1,), (0,))) + dec_b3_ref[...]

    x = x_ref[...]
    z_ref[...] = _dot(x, enc_w_ref[...], ((1,), (0,))) + enc_b_ref[...]
    m = _dot(x, lct_s[...], ((1,), (0,)))
    d = c2_s[...] - 2.0 * m
    v = d.shape[1]
    dmin = jnp.min(d, axis=1, keepdims=True)
    ids = lax.broadcasted_iota(jnp.int32, d.shape, 1)
    tok_ref[0, 0, :] = jnp.min(jnp.where(d <= dmin, ids, jnp.int32(v)), axis=1)


def _make_sc_gather(n_tok, v, e):
    nc, ns = 2, 16
    nw = nc * ns
    tpw = n_tok // nw
    ch = 128
    nch = tpw // ch
    nbuf = 3

    mesh = plsc.VectorSubcoreMesh(core_axis_name="c", subcore_axis_name="s",
                                  num_cores=nc, num_subcores=ns)

    @functools.partial(
        pl.kernel,
        out_type=(jax.ShapeDtypeStruct((n_tok, e), jnp.float32),
                  jax.ShapeDtypeStruct((n_tok * 3,), jnp.float32)),
        mesh=mesh,
        scratch_types=[
            pltpu.VMEM((tpw,), jnp.int32),
            [pltpu.VMEM((ch, e), jnp.float32) for _ in range(nbuf)],
            pltpu.VMEM((v * 3,), jnp.float32),
            pltpu.VMEM((tpw * 3,), jnp.float32),
            [pltpu.SemaphoreType.DMA for _ in range(nbuf)],
        ],
        compiler_params=pltpu.CompilerParams(needs_layout_passes=False),
    )
    def sc_gather(tok_hbm, te_hbm, td_hbm, zq_hbm, rec_hbm,
                  idx_v, gbufs, tdv, recv, gsems):
        w = lax.axis_index("s") * nc + lax.axis_index("c")
        base = w * tpw
        pltpu.sync_copy(tok_hbm.at[pl.ds(base, tpw)], idx_v)
        pltpu.sync_copy(td_hbm, tdv)

        depth = nbuf - 1
        gh = [None] * nbuf
        for k in range(depth):
            gh[k] = pltpu.async_copy(
                te_hbm.at[idx_v.at[pl.ds(k * ch, ch)]], gbufs[k], gsems[k])
        for k in range(nch):
            b = k % nbuf
            if k + depth < nch:
                bn = (k + depth) % nbuf
                gh[bn] = pltpu.async_copy(
                    te_hbm.at[idx_v.at[pl.ds((k + depth) * ch, ch)]],
                    gbufs[bn], gsems[bn])
            gh[b].wait()
            pltpu.sync_copy(gbufs[b], zq_hbm.at[pl.ds(base + k * ch, ch)])

        def rec_group(g, carry):
            idx = idx_v[pl.ds(g * 16, 16)]
            f = idx * 3
            p = (g * 16 + lax.broadcasted_iota(jnp.int32, (16,), 0)) * 3
            plsc.store_scatter(recv, [p], plsc.load_gather(tdv, [f]))
            plsc.store_scatter(recv, [p + 1], plsc.load_gather(tdv, [f + 1]))
            plsc.store_scatter(recv, [p + 2], plsc.load_gather(tdv, [f + 2]))
            return carry

        lax.fori_loop(0, tpw // 16, rec_group, 0)
        pltpu.sync_copy(recv, rec_hbm.at[pl.ds(w * tpw * 3, tpw * 3)])

    return sc_gather


def kernel(x, codebook_w, enc_w, enc_b, cm_w1, cm_b1, cm_g1, cm_be1,
           cm_w2, cm_b2, cm_g2, cm_be2, cm_w3, cm_b3,
           dec_w1, dec_b1, dec_w2, dec_b2, dec_w3, dec_b3):
    b, in_dim = x.shape
    v, d = codebook_w.shape
    e = enc_w.shape[1]
    e2 = dec_w1.shape[1]
    k_tok = in_dim // d
    n = b * k_tok

    x_flat = x.reshape(n, d)
    row = lambda a: a.reshape(1, -1)
    full = lambda shape: pl.BlockSpec(shape, lambda i: tuple(0 for _ in shape))

    grid = 16
    t = n // grid
    z, tok3, te, td = pl.pallas_call(
        _main_body,
        grid=(grid,),
        in_specs=[pl.BlockSpec((t, d), lambda i: (i, 0)),
                  full((v, d)),
                  full((d, e)), full((1, e)), full((1, e)), full((1, e)),
                  full((e, e)), full((1, e)), full((1, e)), full((1, e)),
                  full((e, d)), full((d, 1)),
                  full((d, e)), full((1, e)),
                  full((e, e2)), full((1, e2)),
                  full((e2, e2)), full((1, e2)),
                  full((e2, d)), full((1, d))],
        out_specs=(pl.BlockSpec((t, e), lambda i: (i, 0)),
                   pl.BlockSpec((1, 1, t), lambda i: (i, 0, 0)),
                   full((v, e)),
                   full((v, d))),
        out_shape=(jax.ShapeDtypeStruct((n, e), jnp.float32),
                   jax.ShapeDtypeStruct((grid, 1, t), jnp.int32),
                   jax.ShapeDtypeStruct((v, e), jnp.float32),
                   jax.ShapeDtypeStruct((v, d), jnp.float32)),
        scratch_shapes=[pltpu.VMEM((d, v), jnp.float32),
                        pltpu.VMEM((1, v), jnp.float32)],
    )(x_flat, codebook_w,
      cm_w1, row(cm_b1), row(cm_g1), row(cm_be1),
      cm_w2, row(cm_b2), row(cm_g2), row(cm_be2),
      cm_w3, cm_b3.reshape(-1, 1),
      enc_w, row(enc_b),
      dec_w1, row(dec_b1), dec_w2, row(dec_b2), dec_w3, row(dec_b3))

    tokens = tok3.reshape(n)
    zq, rec_flat = _make_sc_gather(n, v, e)(tokens, te, td.reshape(v * 3))

    return (z.reshape(b, k_tok, e),
            zq.reshape(b, k_tok, e),
            rec_flat.reshape(b, in_dim))


# speedup vs baseline: 1.1669x; 1.0010x over previous
import functools

import jax
import jax.numpy as jnp
from jax import lax
from jax.experimental import pallas as pl
from jax.experimental.pallas import tpu as pltpu
from jax.experimental.pallas import tpu_sc as plsc


def _ln(h, g, b):
    m = jnp.mean(h, axis=-1, keepdims=True)
    v = jnp.var(h, axis=-1, keepdims=True)
    return (h - m) / jnp.sqrt(v + 1e-5) * g + b


def _silu(h):
    return h * jax.nn.sigmoid(h)


def _dot(a, b, dims):
    return lax.dot_general(a, b, (dims, ((), ())),
                           preferred_element_type=jnp.float32)


def _main_body(x_ref, cb_ref, cm_w1_ref, cm_b1_ref, cm_g1_ref, cm_be1_ref,
               cm_w2_ref, cm_b2_ref, cm_g2_ref, cm_be2_ref,
               cm_w3_ref, cm_b3c_ref, enc_w_ref, enc_b_ref,
               dec_w1_ref, dec_b1_ref, dec_w2_ref, dec_b2_ref,
               dec_w3_ref, dec_b3_ref,
               z_ref, tok_ref, te_ref, td_ref,
               lct_s, c2_s):
    i = pl.program_id(0)

    @pl.when(i == 0)
    def _tables():
        cb = cb_ref[...]
        h = _dot(cb, cm_w1_ref[...], ((1,), (0,)))
        h = _silu(_ln(h + cm_b1_ref[...], cm_g1_ref[...], cm_be1_ref[...]))
        h = _dot(h, cm_w2_ref[...], ((1,), (0,)))
        h = _silu(_ln(h + cm_b2_ref[...], cm_g2_ref[...], cm_be2_ref[...]))
        lct = _dot(cm_w3_ref[...], h, ((0,), (1,))) + cm_b3c_ref[...]
        lct_s[...] = lct
        c2_s[...] = jnp.sum(lct * lct, axis=0, keepdims=True)
        te = _dot(lct, enc_w_ref[...], ((0,), (0,))) + enc_b_ref[...]
        te_ref[...] = te
        hd = _silu(_dot(te, dec_w1_ref[...], ((1,), (0,))) + dec_b1_ref[...])
        hd = _silu(_dot(hd, dec_w2_ref[...], ((1,), (0,))) + dec_b2_ref[...])
        td_ref[...] = _dot(hd, dec_w3_ref[...], ((1,), (0,))) + dec_b3_ref[...]

    x = x_ref[...]
    z_ref[...] = _dot(x, enc_w_ref[...], ((1,), (0,))) + enc_b_ref[...]
    m = _dot(x, lct_s[...], ((1,), (0,)))
    d = c2_s[...] - 2.0 * m
    v = d.shape[1]
    dmin = jnp.min(d, axis=1, keepdims=True)
    ids = lax.broadcasted_iota(jnp.int32, d.shape, 1)
    tok_ref[0, 0, :] = jnp.min(jnp.where(d <= dmin, ids, jnp.int32(v)), axis=1)


def _make_sc_gather(n_tok, v, e):
    nc, ns = 2, 16
    nw = nc * ns
    tpw = n_tok // nw
    ch = 128
    nch = tpw // ch
    nbuf = 3

    mesh = plsc.VectorSubcoreMesh(core_axis_name="c", subcore_axis_name="s",
                                  num_cores=nc, num_subcores=ns)

    @functools.partial(
        pl.kernel,
        out_type=(jax.ShapeDtypeStruct((n_tok, e), jnp.float32),
                  jax.ShapeDtypeStruct((n_tok * 3,), jnp.float32)),
        mesh=mesh,
        scratch_types=[
            pltpu.VMEM((tpw,), jnp.int32),
            [pltpu.VMEM((ch, e), jnp.float32) for _ in range(nbuf)],
            pltpu.VMEM((v * 3,), jnp.float32),
            pltpu.VMEM((tpw * 3,), jnp.float32),
            [pltpu.SemaphoreType.DMA for _ in range(nbuf)],
        ],
        compiler_params=pltpu.CompilerParams(needs_layout_passes=False),
    )
    def sc_gather(tok_hbm, te_hbm, td_hbm, zq_hbm, rec_hbm,
                  idx_v, gbufs, tdv, recv, gsems):
        w = lax.axis_index("s") * nc + lax.axis_index("c")
        base = w * tpw
        pltpu.sync_copy(tok_hbm.at[pl.ds(base, tpw)], idx_v)
        pltpu.sync_copy(td_hbm, tdv)

        depth = nbuf - 1
        gh = [None] * nbuf
        for k in range(depth):
            gh[k] = pltpu.async_copy(
                te_hbm.at[idx_v.at[pl.ds(k * ch, ch)]], gbufs[k], gsems[k])
        for k in range(nch):
            b = k % nbuf
            if k + depth < nch:
                bn = (k + depth) % nbuf
                gh[bn] = pltpu.async_copy(
                    te_hbm.at[idx_v.at[pl.ds((k + depth) * ch, ch)]],
                    gbufs[bn], gsems[bn])
            gh[b].wait()
            pltpu.sync_copy(gbufs[b], zq_hbm.at[pl.ds(base + k * ch, ch)])

        def rec_group(g, carry):
            idx = idx_v[pl.ds(g * 16, 16)]
            f = idx * 3
            p = (g * 16 + lax.broadcasted_iota(jnp.int32, (16,), 0)) * 3
            plsc.store_scatter(recv, [p], plsc.load_gather(tdv, [f]))
            plsc.store_scatter(recv, [p + 1], plsc.load_gather(tdv, [f + 1]))
            plsc.store_scatter(recv, [p + 2], plsc.load_gather(tdv, [f + 2]))
            return carry

        lax.fori_loop(0, tpw // 16, rec_group, 0)
        pltpu.sync_copy(recv, rec_hbm.at[pl.ds(w * tpw * 3, tpw * 3)])

    return sc_gather


def kernel(x, codebook_w, enc_w, enc_b, cm_w1, cm_b1, cm_g1, cm_be1,
           cm_w2, cm_b2, cm_g2, cm_be2, cm_w3, cm_b3,
           dec_w1, dec_b1, dec_w2, dec_b2, dec_w3, dec_b3):
    b, in_dim = x.shape
    v, d = codebook_w.shape
    e = enc_w.shape[1]
    e2 = dec_w1.shape[1]
    k_tok = in_dim // d
    n = b * k_tok

    x_flat = x.reshape(n, d)
    row = lambda a: a.reshape(1, -1)
    full = lambda shape: pl.BlockSpec(shape, lambda i: tuple(0 for _ in shape))

    grid = 8
    t = n // grid
    z, tok3, te, td = pl.pallas_call(
        _main_body,
        grid=(grid,),
        in_specs=[pl.BlockSpec((t, d), lambda i: (i, 0)),
                  full((v, d)),
                  full((d, e)), full((1, e)), full((1, e)), full((1, e)),
                  full((e, e)), full((1, e)), full((1, e)), full((1, e)),
                  full((e, d)), full((d, 1)),
                  full((d, e)), full((1, e)),
                  full((e, e2)), full((1, e2)),
                  full((e2, e2)), full((1, e2)),
                  full((e2, d)), full((1, d))],
        out_specs=(pl.BlockSpec((t, e), lambda i: (i, 0)),
                   pl.BlockSpec((1, 1, t), lambda i: (i, 0, 0)),
                   full((v, e)),
                   full((v, d))),
        out_shape=(jax.ShapeDtypeStruct((n, e), jnp.float32),
                   jax.ShapeDtypeStruct((grid, 1, t), jnp.int32),
                   jax.ShapeDtypeStruct((v, e), jnp.float32),
                   jax.ShapeDtypeStruct((v, d), jnp.float32)),
        scratch_shapes=[pltpu.VMEM((d, v), jnp.float32),
                        pltpu.VMEM((1, v), jnp.float32)],
    )(x_flat, codebook_w,
      cm_w1, row(cm_b1), row(cm_g1), row(cm_be1),
      cm_w2, row(cm_b2), row(cm_g2), row(cm_be2),
      cm_w3, cm_b3.reshape(-1, 1),
      enc_w, row(enc_b),
      dec_w1, row(dec_b1), dec_w2, row(dec_b2), dec_w3, row(dec_b3))

    tokens = tok3.reshape(n)
    zq, rec_flat = _make_sc_gather(n, v, e)(tokens, te, td.reshape(v * 3))

    return (z.reshape(b, k_tok, e),
            zq.reshape(b, k_tok, e),
            rec_flat.reshape(b, in_dim))
